# baseline (speedup 1.0000x reference)
import functools

import jax
import jax.numpy as jnp
from jax import lax
from jax.experimental import pallas as pl
from jax.experimental import layout as jex_layout
from jax.experimental.pallas import tpu as pltpu
from jax.experimental.pallas import tpu_sc as plsc

_VOCAB = 1000000
_HID = 32
_B = 4096
_L = 50
_N = _B * _L

_NC = 2
_NS = 16
_NW = _NC * _NS
_PER_W = _N // _NW
_CH = 128
_NCH = _PER_W // _CH
_LANES = 16

_TCOLS_FULL = _VOCAB // 128
_TPW = _TCOLS_FULL // _NW
_SCR_ROWS = _VOCAB // 4
_TAIL = _VOCAB - _TCOLS_FULL * 128


def _tr_body(tabT_hbm, tail_hbm, scr_hbm, vsrc_a, vsrc_b, dst_a, dst_b,
             sem_ia, sem_ib, sem_oa, sem_ob):
    wid = lax.axis_index("s") * _NC + lax.axis_index("c")

    def issue_in(tc, vsrc, sem):
        pltpu.async_copy(tabT_hbm.at[:, pl.ds(tc * 128, 128)], vsrc, sem)

    def wait_in(tc, vsrc, sem):
        pltpu.make_async_copy(tabT_hbm.at[:, pl.ds(tc * 128, 128)], vsrc, sem).wait()

    def issue_out(tc, dst, sem):
        pltpu.async_copy(dst, scr_hbm.at[pl.ds(tc * 32, 32)], sem)

    def wait_out(tc, dst, sem):
        pltpu.make_async_copy(dst, scr_hbm.at[pl.ds(tc * 32, 32)], sem).wait()

    iot = lax.iota(jnp.int32, _LANES)

    def transpose(vsrc, dst):
        def qstep(q, _):
            for k in range(8):
                rowv = iot + 16 * (k % 2)
                colv = jnp.full((_LANES,), q * 4 + k // 2, jnp.int32)
                a = plsc.load_gather(vsrc, [rowv, colv])
                dst[q, pl.ds(16 * k, 16)] = a
            return 0

        lax.fori_loop(0, 32, qstep, 0)

    issue_in(wid, vsrc_a, sem_ia)

    def pair(tp, _):
        t0 = tp * 2
        t1 = t0 + 1
        tc0 = wid + 32 * t0
        tc1 = wid + 32 * t1
        issue_in(tc1, vsrc_b, sem_ib)

        @pl.when(tp > 0)
        def _():
            wait_out(wid + 32 * (t0 - 2), dst_a, sem_oa)

        wait_in(tc0, vsrc_a, sem_ia)
        transpose(vsrc_a, dst_a)
        issue_out(tc0, dst_a, sem_oa)

        @pl.when(tp < _TPW // 2 - 1)
        def _():
            issue_in(wid + 32 * (t0 + 2), vsrc_a, sem_ia)

        @pl.when(tp > 0)
        def _():
            wait_out(wid + 32 * (t1 - 2), dst_b, sem_ob)

        wait_in(tc1, vsrc_b, sem_ib)
        transpose(vsrc_b, dst_b)
        issue_out(tc1, dst_b, sem_ob)
        return 0

    lax.fori_loop(0, _TPW // 2, pair, 0)
    wait_out(wid + 32 * (_TPW - 2), dst_a, sem_oa)
    wait_out(wid + 32 * (_TPW - 1), dst_b, sem_ob)

    @pl.when(wid < 4)
    def _():
        tc = _TCOLS_FULL - 4 + wid
        pltpu.sync_copy(tabT_hbm.at[:, pl.ds(tc * 128, 128)], vsrc_a)
        transpose(vsrc_a, dst_a)
        pltpu.sync_copy(dst_a, scr_hbm.at[pl.ds(tc * 32, 32)])

    @pl.when(wid == 4)
    def _():
        pltpu.sync_copy(tail_hbm, dst_b.at[pl.ds(0, _TAIL // 4)])
        pltpu.sync_copy(dst_b.at[pl.ds(0, _TAIL // 4)],
                        scr_hbm.at[pl.ds(_TCOLS_FULL * 32, _TAIL // 4)])


def _sc_body(inp_hbm, tab_hbm, ft_hbm, out_hbm,
             x_v, ii_v, fi_v, rows_ia, rows_ib, rows_fa, rows_fb,
             out_a, out_b,
             sem_gia, sem_gib, sem_gfa, sem_gfb, sem_oa, sem_ob):
    wid = lax.axis_index("s") * _NC + lax.axis_index("c")
    base = wid * _PER_W

    pltpu.sync_copy(inp_hbm.at[pl.ds(base, _PER_W)], x_v)

    def idx_chunk(c, _):
        for k in range(_CH // _LANES):
            x = x_v[pl.ds(c * _CH + k * _LANES, _LANES)]
            ii = x.astype(jnp.int32)
            fr = x - ii.astype(jnp.float32)
            fi = (fr * 100.0).astype(jnp.int32)
            ii_v[c, pl.ds(k * _LANES, _LANES)] = ii
            fi_v[c, pl.ds(k * _LANES, _LANES)] = fi
        return 0

    lax.fori_loop(0, _NCH, idx_chunk, 0)

    def issue_gathers(c, rows_i, rows_f, sem_i, sem_f):
        pltpu.async_copy(tab_hbm.at[ii_v.at[c]], rows_i, sem_i)
        pltpu.async_copy(ft_hbm.at[fi_v.at[c]], rows_f, sem_f)

    def wait_gathers(c, rows_i, rows_f, sem_i, sem_f):
        pltpu.make_async_copy(tab_hbm.at[ii_v.at[c]], rows_i, sem_i).wait()
        pltpu.make_async_copy(ft_hbm.at[fi_v.at[c]], rows_f, sem_f).wait()

    def add_chunk(rows_i, rows_f, outb):
        def add_rows(j, _):
            for u in range(4):
                jj = j * 4 + u
                for h in range(_HID // _LANES):
                    a = rows_i[jj, pl.ds(h * _LANES, _LANES)]
                    b = rows_f[jj, pl.ds(h * _LANES, _LANES)]
                    outb[jj, pl.ds(h * _LANES, _LANES)] = a + b
            return 0

        lax.fori_loop(0, _CH // 4, add_rows, 0)

    def issue_store(c, outb, sem):
        pltpu.async_copy(outb, out_hbm.at[pl.ds(base + c * _CH, _CH)], sem)

    def wait_store(c, outb, sem):
        pltpu.make_async_copy(outb, out_hbm.at[pl.ds(base + c * _CH, _CH)], sem).wait()

    issue_gathers(0, rows_ia, rows_fa, sem_gia, sem_gfa)

    def pair(cp, _):
        c0 = cp * 2
        c1 = c0 + 1
        issue_gathers(c1, rows_ib, rows_fb, sem_gib, sem_gfb)

        @pl.when(cp > 0)
        def _():
            wait_store(c0 - 2, out_a, sem_oa)

        wait_gathers(c0, rows_ia, rows_fa, sem_gia, sem_gfa)
        add_chunk(rows_ia, rows_fa, out_a)
        issue_store(c0, out_a, sem_oa)

        @pl.when(cp < _NCH // 2 - 1)
        def _():
            issue_gathers(c0 + 2, rows_ia, rows_fa, sem_gia, sem_gfa)

        @pl.when(cp > 0)
        def _():
            wait_store(c1 - 2, out_b, sem_ob)

        wait_gathers(c1, rows_ib, rows_fb, sem_gib, sem_gfb)
        add_chunk(rows_ib, rows_fb, out_b)
        issue_store(c1, out_b, sem_ob)
        return 0

    lax.fori_loop(0, _NCH // 2, pair, 0)
    wait_store(_NCH - 2, out_a, sem_oa)
    wait_store(_NCH - 1, out_b, sem_ob)


@functools.partial(jax.jit)
def kernel(input, int_table, float_table):
    mesh = plsc.VectorSubcoreMesh(core_axis_name="c", subcore_axis_name="s")
    flat = input.reshape(_N)

    tr_call = pl.kernel(
        _tr_body,
        out_type=jax.ShapeDtypeStruct((_SCR_ROWS, 128), jnp.float32),
        mesh=mesh,
        compiler_params=pltpu.CompilerParams(
            use_tc_tiling_on_sc=True, needs_layout_passes=False),
        scratch_types=[
            pltpu.VMEM((_HID, 128), jnp.float32),
            pltpu.VMEM((_HID, 128), jnp.float32),
            pltpu.VMEM((32, 128), jnp.float32),
            pltpu.VMEM((32, 128), jnp.float32),
            pltpu.SemaphoreType.DMA,
            pltpu.SemaphoreType.DMA,
            pltpu.SemaphoreType.DMA,
            pltpu.SemaphoreType.DMA,
        ],
    )
    tail = lax.slice(int_table, (_TCOLS_FULL * 128, 0), (_VOCAB, _HID))
    scr = tr_call(int_table.T, tail.reshape(_TAIL // 4, 128))
    tabp = scr.reshape(4 * _SCR_ROWS, _HID)

    sc_call = pl.kernel(
        _sc_body,
        out_type=jax.ShapeDtypeStruct((_N, _HID), jnp.float32),
        mesh=mesh,
        compiler_params=pltpu.CompilerParams(use_tc_tiling_on_sc=False),
        scratch_types=[
            pltpu.VMEM((_PER_W,), jnp.float32),
            pltpu.VMEM((_NCH, _CH), jnp.int32),
            pltpu.VMEM((_NCH, _CH), jnp.int32),
            pltpu.VMEM((_CH, _HID), jnp.float32),
            pltpu.VMEM((_CH, _HID), jnp.float32),
            pltpu.VMEM((_CH, _HID), jnp.float32),
            pltpu.VMEM((_CH, _HID), jnp.float32),
            pltpu.VMEM((_CH, _HID), jnp.float32),
            pltpu.VMEM((_CH, _HID), jnp.float32),
            pltpu.SemaphoreType.DMA,
            pltpu.SemaphoreType.DMA,
            pltpu.SemaphoreType.DMA,
            pltpu.SemaphoreType.DMA,
            pltpu.SemaphoreType.DMA,
            pltpu.SemaphoreType.DMA,
        ],
    )
    out_flat = sc_call(flat, tabp, float_table)
    out = out_flat.reshape(_B, _L, _HID)
    return jex_layout.with_layout_constraint(out, jex_layout.Layout((0, 1, 2)))


# speedup vs baseline: 1.8623x; 1.8623x over previous
import functools

import jax
import jax.numpy as jnp
from jax import lax
from jax.experimental import pallas as pl
from jax.experimental import layout as jex_layout
from jax.experimental.pallas import tpu as pltpu
from jax.experimental.pallas import tpu_sc as plsc

_VOCAB = 1000000
_HID = 32
_B = 4096
_L = 50
_N = _B * _L

_NC = 2
_NS = 16
_NW = _NC * _NS
_PER_W = _N // _NW
_CH = 128
_NCH = _PER_W // _CH
_LANES = 16

_TCOLS_FULL = _VOCAB // 128
_TPW = _TCOLS_FULL // _NW
_SCR_ROWS = _VOCAB // 4
_TAIL = _VOCAB - _TCOLS_FULL * 128


def _tr_body(tabT_hbm, tail_hbm, scr_hbm, vsrc_a, vsrc_b, dst_a, dst_b,
             sem_ia, sem_ib, sem_oa, sem_ob):
    wid = lax.axis_index("s") * _NC + lax.axis_index("c")

    def issue_in(tc, vsrc, sem):
        pltpu.async_copy(tabT_hbm.at[:, pl.ds(tc * 128, 128)], vsrc, sem)

    def wait_in(tc, vsrc, sem):
        pltpu.make_async_copy(tabT_hbm.at[:, pl.ds(tc * 128, 128)], vsrc, sem).wait()

    def issue_out(tc, dst, sem):
        pltpu.async_copy(dst, scr_hbm.at[pl.ds(tc * 32, 32)], sem)

    def wait_out(tc, dst, sem):
        pltpu.make_async_copy(dst, scr_hbm.at[pl.ds(tc * 32, 32)], sem).wait()

    iot = lax.iota(jnp.int32, _LANES)

    def transpose(vsrc, dst):
        def hstep(h0, _):
            hvec = (h0 + iot) & 31
            for k in range(8):
                c0 = k * _LANES
                cvec = iot + c0
                qvec = lax.shift_right_logical(iot, 2) + (c0 // 4)
                wvec = (iot & 3) * _HID + hvec
                a = plsc.load_gather(vsrc, [hvec, cvec])
                plsc.store_scatter(dst, [qvec, wvec], a)
            return 0

        lax.fori_loop(0, 32, hstep, 0)

    issue_in(wid, vsrc_a, sem_ia)

    def pair(tp, _):
        t0 = tp * 2
        t1 = t0 + 1
        tc0 = wid + 32 * t0
        tc1 = wid + 32 * t1
        issue_in(tc1, vsrc_b, sem_ib)

        @pl.when(tp > 0)
        def _():
            wait_out(wid + 32 * (t0 - 2), dst_a, sem_oa)

        wait_in(tc0, vsrc_a, sem_ia)
        transpose(vsrc_a, dst_a)
        issue_out(tc0, dst_a, sem_oa)

        @pl.when(tp < _TPW // 2 - 1)
        def _():
            issue_in(wid + 32 * (t0 + 2), vsrc_a, sem_ia)

        @pl.when(tp > 0)
        def _():
            wait_out(wid + 32 * (t1 - 2), dst_b, sem_ob)

        wait_in(tc1, vsrc_b, sem_ib)
        transpose(vsrc_b, dst_b)
        issue_out(tc1, dst_b, sem_ob)
        return 0

    lax.fori_loop(0, _TPW // 2, pair, 0)
    wait_out(wid + 32 * (_TPW - 2), dst_a, sem_oa)
    wait_out(wid + 32 * (_TPW - 1), dst_b, sem_ob)

    @pl.when(wid < 4)
    def _():
        tc = _TCOLS_FULL - 4 + wid
        pltpu.sync_copy(tabT_hbm.at[:, pl.ds(tc * 128, 128)], vsrc_a)
        transpose(vsrc_a, dst_a)
        pltpu.sync_copy(dst_a, scr_hbm.at[pl.ds(tc * 32, 32)])

    @pl.when(wid == 4)
    def _():
        pltpu.sync_copy(tail_hbm, dst_b.at[pl.ds(0, _TAIL // 4)])
        pltpu.sync_copy(dst_b.at[pl.ds(0, _TAIL // 4)],
                        scr_hbm.at[pl.ds(_TCOLS_FULL * 32, _TAIL // 4)])


def _sc_body(inp_hbm, tab_hbm, ft_hbm, out_hbm,
             x_v, ii_v, fi_v, rows_ia, rows_ib, rows_fa, rows_fb,
             out_a, out_b,
             sem_gia, sem_gib, sem_gfa, sem_gfb, sem_oa, sem_ob):
    wid = lax.axis_index("s") * _NC + lax.axis_index("c")
    base = wid * _PER_W

    pltpu.sync_copy(inp_hbm.at[pl.ds(base, _PER_W)], x_v)

    def idx_chunk(c, _):
        for k in range(_CH // _LANES):
            x = x_v[pl.ds(c * _CH + k * _LANES, _LANES)]
            ii = x.astype(jnp.int32)
            fr = x - ii.astype(jnp.float32)
            fi = (fr * 100.0).astype(jnp.int32)
            ii_v[c, pl.ds(k * _LANES, _LANES)] = ii
            fi_v[c, pl.ds(k * _LANES, _LANES)] = fi
        return 0

    lax.fori_loop(0, _NCH, idx_chunk, 0)

    def issue_gathers(c, rows_i, rows_f, sem_i, sem_f):
        pltpu.async_copy(tab_hbm.at[ii_v.at[c]], rows_i, sem_i)
        pltpu.async_copy(ft_hbm.at[fi_v.at[c]], rows_f, sem_f)

    def wait_gathers(c, rows_i, rows_f, sem_i, sem_f):
        pltpu.make_async_copy(tab_hbm.at[ii_v.at[c]], rows_i, sem_i).wait()
        pltpu.make_async_copy(ft_hbm.at[fi_v.at[c]], rows_f, sem_f).wait()

    def add_chunk(rows_i, rows_f, outb):
        def add_rows(j, _):
            for u in range(4):
                jj = j * 4 + u
                for h in range(_HID // _LANES):
                    a = rows_i[jj, pl.ds(h * _LANES, _LANES)]
                    b = rows_f[jj, pl.ds(h * _LANES, _LANES)]
                    outb[jj, pl.ds(h * _LANES, _LANES)] = a + b
            return 0

        lax.fori_loop(0, _CH // 4, add_rows, 0)

    def issue_store(c, outb, sem):
        pltpu.async_copy(outb, out_hbm.at[pl.ds(base + c * _CH, _CH)], sem)

    def wait_store(c, outb, sem):
        pltpu.make_async_copy(outb, out_hbm.at[pl.ds(base + c * _CH, _CH)], sem).wait()

    issue_gathers(0, rows_ia, rows_fa, sem_gia, sem_gfa)

    def pair(cp, _):
        c0 = cp * 2
        c1 = c0 + 1
        issue_gathers(c1, rows_ib, rows_fb, sem_gib, sem_gfb)

        @pl.when(cp > 0)
        def _():
            wait_store(c0 - 2, out_a, sem_oa)

        wait_gathers(c0, rows_ia, rows_fa, sem_gia, sem_gfa)
        add_chunk(rows_ia, rows_fa, out_a)
        issue_store(c0, out_a, sem_oa)

        @pl.when(cp < _NCH // 2 - 1)
        def _():
            issue_gathers(c0 + 2, rows_ia, rows_fa, sem_gia, sem_gfa)

        @pl.when(cp > 0)
        def _():
            wait_store(c1 - 2, out_b, sem_ob)

        wait_gathers(c1, rows_ib, rows_fb, sem_gib, sem_gfb)
        add_chunk(rows_ib, rows_fb, out_b)
        issue_store(c1, out_b, sem_ob)
        return 0

    lax.fori_loop(0, _NCH // 2, pair, 0)
    wait_store(_NCH - 2, out_a, sem_oa)
    wait_store(_NCH - 1, out_b, sem_ob)


@functools.partial(jax.jit)
def kernel(input, int_table, float_table):
    mesh = plsc.VectorSubcoreMesh(core_axis_name="c", subcore_axis_name="s")
    flat = input.reshape(_N)

    tr_call = pl.kernel(
        _tr_body,
        out_type=jax.ShapeDtypeStruct((_SCR_ROWS, 128), jnp.float32),
        mesh=mesh,
        compiler_params=pltpu.CompilerParams(
            use_tc_tiling_on_sc=True, needs_layout_passes=False),
        scratch_types=[
            pltpu.VMEM((_HID, 128), jnp.float32),
            pltpu.VMEM((_HID, 128), jnp.float32),
            pltpu.VMEM((32, 128), jnp.float32),
            pltpu.VMEM((32, 128), jnp.float32),
            pltpu.SemaphoreType.DMA,
            pltpu.SemaphoreType.DMA,
            pltpu.SemaphoreType.DMA,
            pltpu.SemaphoreType.DMA,
        ],
    )
    tail = lax.slice(int_table, (_TCOLS_FULL * 128, 0), (_VOCAB, _HID))
    scr = tr_call(int_table.T, tail.reshape(_TAIL // 4, 128))
    tabp = scr.reshape(4 * _SCR_ROWS, _HID)

    sc_call = pl.kernel(
        _sc_body,
        out_type=jax.ShapeDtypeStruct((_N, _HID), jnp.float32),
        mesh=mesh,
        compiler_params=pltpu.CompilerParams(use_tc_tiling_on_sc=False),
        scratch_types=[
            pltpu.VMEM((_PER_W,), jnp.float32),
            pltpu.VMEM((_NCH, _CH), jnp.int32),
            pltpu.VMEM((_NCH, _CH), jnp.int32),
            pltpu.VMEM((_CH, _HID), jnp.float32),
            pltpu.VMEM((_CH, _HID), jnp.float32),
            pltpu.VMEM((_CH, _HID), jnp.float32),
            pltpu.VMEM((_CH, _HID), jnp.float32),
            pltpu.VMEM((_CH, _HID), jnp.float32),
            pltpu.VMEM((_CH, _HID), jnp.float32),
            pltpu.SemaphoreType.DMA,
            pltpu.SemaphoreType.DMA,
            pltpu.SemaphoreType.DMA,
            pltpu.SemaphoreType.DMA,
            pltpu.SemaphoreType.DMA,
            pltpu.SemaphoreType.DMA,
        ],
    )
    out_flat = sc_call(flat, tabp, float_table)
    out = out_flat.reshape(_B, _L, _HID)
    return jex_layout.with_layout_constraint(out, jex_layout.Layout((0, 1, 2)))


# speedup vs baseline: 2.2823x; 1.2255x over previous
import functools

import jax
import jax.numpy as jnp
from jax import lax
from jax.experimental import pallas as pl
from jax.experimental.pallas import tpu as pltpu
from jax.experimental.pallas import tpu_sc as plsc

_VOCAB = 1000000
_HID = 32
_B = 4096
_L = 50
_N = _B * _L

_NC = 2
_NS = 16
_NW = _NC * _NS
_BPW = _B // _NW
_CH = 128
_LANES = 16
_DEPTH = 4

_TCOLS_FULL = _VOCAB // 128
_TPW = _TCOLS_FULL // _NW
_SCR_ROWS = _VOCAB // 4
_TAIL = _VOCAB - _TCOLS_FULL * 128
_OUT_ROWS = _N * _HID // 128


def _tr_body(tabT_hbm, tail_hbm, scr_hbm, vsrc_a, vsrc_b, dst_a, dst_b,
             sem_ia, sem_ib, sem_oa, sem_ob):
    wid = lax.axis_index("s") * _NC + lax.axis_index("c")

    def issue_in(tc, vsrc, sem):
        pltpu.async_copy(tabT_hbm.at[:, pl.ds(tc * 128, 128)], vsrc, sem)

    def wait_in(tc, vsrc, sem):
        pltpu.make_async_copy(tabT_hbm.at[:, pl.ds(tc * 128, 128)], vsrc, sem).wait()

    def issue_out(tc, dst, sem):
        pltpu.async_copy(dst, scr_hbm.at[pl.ds(tc * 32, 32)], sem)

    def wait_out(tc, dst, sem):
        pltpu.make_async_copy(dst, scr_hbm.at[pl.ds(tc * 32, 32)], sem).wait()

    iot = lax.iota(jnp.int32, _LANES)

    def transpose(vsrc, dst):
        def hstep(h0, _):
            hvec = (h0 + iot) & 31
            for k in range(8):
                c0 = k * _LANES
                cvec = iot + c0
                qvec = lax.shift_right_logical(iot, 2) + (c0 // 4)
                wvec = (iot & 3) * _HID + hvec
                a = plsc.load_gather(vsrc, [hvec, cvec])
                plsc.store_scatter(dst, [qvec, wvec], a)
            return 0

        lax.fori_loop(0, 32, hstep, 0)

    issue_in(wid, vsrc_a, sem_ia)

    def pair(tp, _):
        t0 = tp * 2
        t1 = t0 + 1
        tc0 = wid + 32 * t0
        tc1 = wid + 32 * t1
        issue_in(tc1, vsrc_b, sem_ib)

        @pl.when(tp > 0)
        def _():
            wait_out(wid + 32 * (t0 - 2), dst_a, sem_oa)

        wait_in(tc0, vsrc_a, sem_ia)
        transpose(vsrc_a, dst_a)
        issue_out(tc0, dst_a, sem_oa)

        @pl.when(tp < _TPW // 2 - 1)
        def _():
            issue_in(wid + 32 * (t0 + 2), vsrc_a, sem_ia)

        @pl.when(tp > 0)
        def _():
            wait_out(wid + 32 * (t1 - 2), dst_b, sem_ob)

        wait_in(tc1, vsrc_b, sem_ib)
        transpose(vsrc_b, dst_b)
        issue_out(tc1, dst_b, sem_ob)
        return 0

    lax.fori_loop(0, _TPW // 2, pair, 0)
    wait_out(wid + 32 * (_TPW - 2), dst_a, sem_oa)
    wait_out(wid + 32 * (_TPW - 1), dst_b, sem_ob)

    @pl.when(wid < 4)
    def _():
        tc = _TCOLS_FULL - 4 + wid
        pltpu.sync_copy(tabT_hbm.at[:, pl.ds(tc * 128, 128)], vsrc_a)
        transpose(vsrc_a, dst_a)
        pltpu.sync_copy(dst_a, scr_hbm.at[pl.ds(tc * 32, 32)])

    @pl.when(wid == 4)
    def _():
        pltpu.sync_copy(tail_hbm, dst_b.at[pl.ds(0, _TAIL // 4)])
        pltpu.sync_copy(dst_b.at[pl.ds(0, _TAIL // 4)],
                        scr_hbm.at[pl.ds(_TCOLS_FULL * 32, _TAIL // 4)])


def _sc_body(inp_hbm, tab_hbm, ft_hbm, out_hbm,
             x_v, ii_v, fi_v,
             ri0, ri1, ri2, ri3, rf0, rf1, rf2, rf3, ot0, ot1, ot2, ot3,
             sgi0, sgi1, sgi2, sgi3, sgf0, sgf1, sgf2, sgf3,
             so0, so1, so2, so3):
    wid = lax.axis_index("s") * _NC + lax.axis_index("c")
    base = wid * _BPW * _L
    iot = lax.iota(jnp.int32, _LANES)

    rows_i = (ri0, ri1, ri2, ri3)
    rows_f = (rf0, rf1, rf2, rf3)
    outs = (ot0, ot1, ot2, ot3)
    sem_gi = (sgi0, sgi1, sgi2, sgi3)
    sem_gf = (sgf0, sgf1, sgf2, sgf3)
    sem_o = (so0, so1, so2, so3)

    pltpu.sync_copy(inp_hbm.at[pl.ds(base, _BPW * _L)], x_v)

    pos0 = iot * _L

    def idx_chunk(l, _):
        for g in range(_CH // _LANES):
            x = plsc.load_gather(x_v, [pos0 + (g * _LANES * _L + l)])
            ii = x.astype(jnp.int32)
            fr = x - ii.astype(jnp.float32)
            fi = (fr * 100.0).astype(jnp.int32)
            ii_v[l, pl.ds(g * _LANES, _LANES)] = ii
            fi_v[l, pl.ds(g * _LANES, _LANES)] = fi
        return 0

    lax.fori_loop(0, _L, idx_chunk, 0)

    def issue_gathers(l, i):
        pltpu.async_copy(tab_hbm.at[ii_v.at[l]], rows_i[i], sem_gi[i])
        pltpu.async_copy(ft_hbm.at[fi_v.at[l]], rows_f[i], sem_gf[i])

    def wait_gathers(l, i):
        pltpu.make_async_copy(tab_hbm.at[ii_v.at[l]], rows_i[i], sem_gi[i]).wait()
        pltpu.make_async_copy(ft_hbm.at[fi_v.at[l]], rows_f[i], sem_gf[i]).wait()

    def combine_chunk(i):
        ri, rf, ot = rows_i[i], rows_f[i], outs[i]

        def hstep(h0, _):
            hvec = (h0 + iot) & 31
            for k in range(8):
                bvec = iot + k * _LANES
                a = plsc.load_gather(ri, [bvec, hvec])
                b = plsc.load_gather(rf, [bvec, hvec])
                plsc.store_scatter(ot, [hvec, bvec], a + b)
            return 0

        lax.fori_loop(0, 32, hstep, 0)

    def issue_stores(l, i):
        for tr in range(4):
            row0 = l * 1024 + tr * 256 + wid * 8
            pltpu.async_copy(outs[i].at[pl.ds(tr * 8, 8)],
                             out_hbm.at[pl.ds(row0, 8)], sem_o[i])

    def wait_stores(l, i):
        for tr in range(4):
            row0 = l * 1024 + tr * 256 + wid * 8
            pltpu.make_async_copy(outs[i].at[pl.ds(tr * 8, 8)],
                                  out_hbm.at[pl.ds(row0, 8)], sem_o[i]).wait()

    def process(l, i, first):
        if not first:
            @pl.when(l >= _DEPTH)
            def _():
                wait_stores(l - _DEPTH, i)
        wait_gathers(l, i)
        combine_chunk(i)
        issue_stores(l, i)

        @pl.when(l + _DEPTH < _L)
        def _():
            issue_gathers(l + _DEPTH, i)

    for i in range(_DEPTH):
        issue_gathers(i, i)

    def quad(qp, _):
        for i in range(_DEPTH):
            process(qp * _DEPTH + i, i, False)
        return 0

    lax.fori_loop(0, _L // _DEPTH, quad, 0)
    for i in range(_L % _DEPTH):
        process(_L - (_L % _DEPTH) + i, i, False)
    for l in range(_L - _DEPTH, _L):
        wait_stores(l, l % _DEPTH)


@functools.partial(jax.jit)
def kernel(input, int_table, float_table):
    mesh = plsc.VectorSubcoreMesh(core_axis_name="c", subcore_axis_name="s")
    flat = input.reshape(_N)

    tr_call = pl.kernel(
        _tr_body,
        out_type=jax.ShapeDtypeStruct((_SCR_ROWS, 128), jnp.float32),
        mesh=mesh,
        compiler_params=pltpu.CompilerParams(
            use_tc_tiling_on_sc=True, needs_layout_passes=False),
        scratch_types=[
            pltpu.VMEM((_HID, 128), jnp.float32),
            pltpu.VMEM((_HID, 128), jnp.float32),
            pltpu.VMEM((32, 128), jnp.float32),
            pltpu.VMEM((32, 128), jnp.float32),
            pltpu.SemaphoreType.DMA,
            pltpu.SemaphoreType.DMA,
            pltpu.SemaphoreType.DMA,
            pltpu.SemaphoreType.DMA,
        ],
    )
    tail = lax.slice(int_table, (_TCOLS_FULL * 128, 0), (_VOCAB, _HID))
    scr = tr_call(int_table.T, tail.reshape(_TAIL // 4, 128))
    tabp = scr.reshape(_VOCAB, _HID)

    sc_call = pl.kernel(
        _sc_body,
        out_type=jax.ShapeDtypeStruct((_OUT_ROWS, 128), jnp.float32),
        mesh=mesh,
        compiler_params=pltpu.CompilerParams(
            use_tc_tiling_on_sc=False, needs_layout_passes=False),
        scratch_types=(
            [pltpu.VMEM((_BPW * _L,), jnp.float32),
             pltpu.VMEM((_L, _CH), jnp.int32),
             pltpu.VMEM((_L, _CH), jnp.int32)]
            + [pltpu.VMEM((_CH, _HID), jnp.float32)] * 8
            + [pltpu.VMEM((_HID, _CH), jnp.float32)] * 4
            + [pltpu.SemaphoreType.DMA] * 12
        ),
    )
    out_lines = sc_call(flat, tabp, float_table)
    out = (out_lines.reshape(_L, 4, _NW, 8, 128)
           .transpose(2, 4, 0, 1, 3)
           .reshape(_B, _L, _HID))
    return out


# speedup vs baseline: 4.6326x; 2.0298x over previous
import functools

import jax
import jax.numpy as jnp
from jax import lax
from jax.experimental import pallas as pl
from jax.experimental.pallas import tpu as pltpu
from jax.experimental.pallas import tpu_sc as plsc

_VOCAB = 1000000
_HID = 32
_B = 4096
_L = 50
_N = _B * _L

_NC = 2
_NS = 16
_NW = _NC * _NS
_BPW = _B // _NW
_CH = 128
_LANES = 16
_DEPTH = 4

_TCOLS_FULL = _VOCAB // 128
_TPW = _TCOLS_FULL // _NW
_SCR_ROWS = _VOCAB // 4
_TAIL = _VOCAB - _TCOLS_FULL * 128
_OUT_ROWS = _N * _HID // 128


_BAT = 4
_STEPS = _TPW // _BAT


def _tr_body(tabT_hbm, tail_hbm, scr_hbm, vsrc_a, vsrc_b, dst_a, dst_b,
             sem_ia, sem_ib, sem_oa, sem_ob):
    wid = lax.axis_index("s") * _NC + lax.axis_index("c")
    tbase = wid * _TPW

    def issue_in(tc, n, vsrc, sem):
        pltpu.async_copy(tabT_hbm.at[:, pl.ds(tc * 128, n * 128)], vsrc, sem)

    def wait_in(tc, n, vsrc, sem):
        pltpu.make_async_copy(
            tabT_hbm.at[:, pl.ds(tc * 128, n * 128)], vsrc, sem).wait()

    def issue_out(tc, n, dst, sem):
        pltpu.async_copy(dst, scr_hbm.at[pl.ds(tc * 32, n * 32)], sem)

    def wait_out(tc, n, dst, sem):
        pltpu.make_async_copy(
            dst, scr_hbm.at[pl.ds(tc * 32, n * 32)], sem).wait()

    iot = lax.iota(jnp.int32, _LANES)

    def transpose(vsrc, dst, n):
        def hstep(h0, _):
            hvec = (h0 + iot) & 31
            wvec = (iot & 3) * _HID + hvec
            qv0 = lax.shift_right_logical(iot, 2)
            loads = []
            for k in range(n * 8):
                c0 = k * _LANES
                loads.append(plsc.load_gather(vsrc, [hvec, iot + c0]))
            for k in range(n * 8):
                c0 = k * _LANES
                plsc.store_scatter(dst, [qv0 + (c0 // 4), wvec], loads[k])
            return 0

        lax.fori_loop(0, 32, hstep, 0)

    issue_in(tbase, _BAT, vsrc_a, sem_ia)

    def pair(tp, _):
        tc0 = tbase + 2 * tp * _BAT
        tc1 = tc0 + _BAT
        issue_in(tc1, _BAT, vsrc_b, sem_ib)

        @pl.when(tp > 0)
        def _():
            wait_out(tc0 - 2 * _BAT, _BAT, dst_a, sem_oa)

        wait_in(tc0, _BAT, vsrc_a, sem_ia)
        transpose(vsrc_a, dst_a, _BAT)
        issue_out(tc0, _BAT, dst_a, sem_oa)

        @pl.when(tp < _STEPS // 2 - 1)
        def _():
            issue_in(tc0 + 2 * _BAT, _BAT, vsrc_a, sem_ia)

        @pl.when(tp > 0)
        def _():
            wait_out(tc1 - 2 * _BAT, _BAT, dst_b, sem_ob)

        wait_in(tc1, _BAT, vsrc_b, sem_ib)
        transpose(vsrc_b, dst_b, _BAT)
        issue_out(tc1, _BAT, dst_b, sem_ob)
        return 0

    lax.fori_loop(0, _STEPS // 2, pair, 0)
    wait_out(tbase + (_STEPS - 2) * _BAT, _BAT, dst_a, sem_oa)
    wait_out(tbase + (_STEPS - 1) * _BAT, _BAT, dst_b, sem_ob)

    tlast = tbase + (_STEPS - 1) * _BAT
    pltpu.sync_copy(tabT_hbm.at[:, pl.ds(tlast * 128, _BAT * 128)], vsrc_a)
    transpose(vsrc_a, dst_a, _BAT)
    pltpu.sync_copy(dst_a, scr_hbm.at[pl.ds(tlast * 32, _BAT * 32)])

    @pl.when(wid < 4)
    def _():
        tc = _TCOLS_FULL - 4 + wid
        pltpu.sync_copy(tabT_hbm.at[:, pl.ds(tc * 128, 128)],
                        vsrc_b.at[:, pl.ds(0, 128)])
        transpose(vsrc_b, dst_b, 1)
        pltpu.sync_copy(dst_b.at[pl.ds(0, 32)], scr_hbm.at[pl.ds(tc * 32, 32)])

    @pl.when(wid == 4)
    def _():
        pltpu.sync_copy(tail_hbm, dst_b.at[pl.ds(0, _TAIL // 4)])
        pltpu.sync_copy(dst_b.at[pl.ds(0, _TAIL // 4)],
                        scr_hbm.at[pl.ds(_TCOLS_FULL * 32, _TAIL // 4)])


def _sc_body(inp_hbm, tab_hbm, ft_hbm, out_hbm,
             x_v, ii_v, fi_v, ft_v,
             ri0, ri1, ri2, ri3, ot0, ot1, ot2, ot3,
             sgi0, sgi1, sgi2, sgi3,
             so0, so1, so2, so3):
    wid = lax.axis_index("s") * _NC + lax.axis_index("c")
    base = wid * _BPW * _L
    iot = lax.iota(jnp.int32, _LANES)

    rows_i = (ri0, ri1, ri2, ri3)
    outs = (ot0, ot1, ot2, ot3)
    sem_gi = (sgi0, sgi1, sgi2, sgi3)
    sem_o = (so0, so1, so2, so3)

    pltpu.sync_copy(inp_hbm.at[pl.ds(base, _BPW * _L)], x_v)
    pltpu.sync_copy(ft_hbm, ft_v)

    pos0 = iot * _L

    def idx_chunk(l, _):
        for g in range(_CH // _LANES):
            x = plsc.load_gather(x_v, [pos0 + (g * _LANES * _L + l)])
            ii = x.astype(jnp.int32)
            fr = x - ii.astype(jnp.float32)
            fi = (fr * 100.0).astype(jnp.int32)
            ii_v[l, pl.ds(g * _LANES, _LANES)] = ii
            fi_v[l, pl.ds(g * _LANES, _LANES)] = fi
        return 0

    lax.fori_loop(0, _L, idx_chunk, 0)

    def issue_gathers(l, i):
        pltpu.async_copy(tab_hbm.at[ii_v.at[l]], rows_i[i], sem_gi[i])

    def wait_gathers(l, i):
        pltpu.make_async_copy(tab_hbm.at[ii_v.at[l]], rows_i[i], sem_gi[i]).wait()

    def combine_chunk(l, i):
        ri, ot = rows_i[i], outs[i]

        for k in range(8):
            bvec = iot + k * _LANES
            fvec = fi_v[l, pl.ds(k * _LANES, _LANES)]

            def hstep(h0, _):
                hvec = (h0 + iot) & 31
                a = plsc.load_gather(ri, [bvec, hvec])
                b = plsc.load_gather(ft_v, [fvec, hvec])
                plsc.store_scatter(ot, [hvec, bvec], a + b)
                return 0

            lax.fori_loop(0, 32, hstep, 0)

    def issue_stores(l, i):
        for tr in range(4):
            row0 = l * 1024 + tr * 256 + wid * 8
            pltpu.async_copy(outs[i].at[pl.ds(tr * 8, 8)],
                             out_hbm.at[pl.ds(row0, 8)], sem_o[i])

    def wait_stores(l, i):
        for tr in range(4):
            row0 = l * 1024 + tr * 256 + wid * 8
            pltpu.make_async_copy(outs[i].at[pl.ds(tr * 8, 8)],
                                  out_hbm.at[pl.ds(row0, 8)], sem_o[i]).wait()

    def process(l, i, first):
        if not first:
            @pl.when(l >= _DEPTH)
            def _():
                wait_stores(l - _DEPTH, i)
        wait_gathers(l, i)
        combine_chunk(l, i)
        issue_stores(l, i)

        @pl.when(l + _DEPTH < _L)
        def _():
            issue_gathers(l + _DEPTH, i)

    for i in range(_DEPTH):
        issue_gathers(i, i)

    def quad(qp, _):
        for i in range(_DEPTH):
            process(qp * _DEPTH + i, i, False)
        return 0

    lax.fori_loop(0, _L // _DEPTH, quad, 0)
    for i in range(_L % _DEPTH):
        process(_L - (_L % _DEPTH) + i, i, False)
    for l in range(_L - _DEPTH, _L):
        wait_stores(l, l % _DEPTH)


@functools.partial(jax.jit)
def kernel(input, int_table, float_table):
    mesh = plsc.VectorSubcoreMesh(core_axis_name="c", subcore_axis_name="s")
    flat = input.reshape(_N)

    tr_call = pl.kernel(
        _tr_body,
        out_type=jax.ShapeDtypeStruct((_SCR_ROWS, 128), jnp.float32),
        mesh=mesh,
        compiler_params=pltpu.CompilerParams(
            use_tc_tiling_on_sc=True, needs_layout_passes=False),
        scratch_types=[
            pltpu.VMEM((_HID, _BAT * 128), jnp.float32),
            pltpu.VMEM((_HID, _BAT * 128), jnp.float32),
            pltpu.VMEM((_BAT * 32, 128), jnp.float32),
            pltpu.VMEM((_BAT * 32, 128), jnp.float32),
            pltpu.SemaphoreType.DMA,
            pltpu.SemaphoreType.DMA,
            pltpu.SemaphoreType.DMA,
            pltpu.SemaphoreType.DMA,
        ],
    )
    tail = lax.slice(int_table, (_TCOLS_FULL * 128, 0), (_VOCAB, _HID))
    scr = tr_call(int_table.T, tail.reshape(_TAIL // 4, 128))
    tabp = scr.reshape(_VOCAB, _HID)

    sc_call = pl.kernel(
        _sc_body,
        out_type=jax.ShapeDtypeStruct((_OUT_ROWS, 128), jnp.float32),
        mesh=mesh,
        compiler_params=pltpu.CompilerParams(
            use_tc_tiling_on_sc=False, needs_layout_passes=False),
        scratch_types=(
            [pltpu.VMEM((_BPW * _L,), jnp.float32),
             pltpu.VMEM((_L, _CH), jnp.int32),
             pltpu.VMEM((_L, _CH), jnp.int32),
             pltpu.VMEM((10 ** 2, _HID), jnp.float32)]
            + [pltpu.VMEM((_CH, _HID), jnp.float32)] * 4
            + [pltpu.VMEM((_HID, _CH), jnp.float32)] * 4
            + [pltpu.SemaphoreType.DMA] * 8
        ),
    )
    out_lines = sc_call(flat, tabp, float_table)
    out = (out_lines.reshape(_L, 4, _NW, 8, 128)
           .transpose(2, 4, 0, 1, 3)
           .reshape(_B, _L, _HID))
    return out


# speedup vs baseline: 4.7282x; 1.0206x over previous
import functools

import jax
import jax.numpy as jnp
from jax import lax
from jax.experimental import pallas as pl
from jax.experimental.pallas import tpu as pltpu
from jax.experimental.pallas import tpu_sc as plsc

_VOCAB = 1000000
_HID = 32
_B = 4096
_L = 50
_N = _B * _L

_NC = 2
_NS = 16
_NW = _NC * _NS
_BPW = _B // _NW
_CH = 128
_LANES = 16
_DEPTH = 4

_TCOLS_FULL = _VOCAB // 128
_TPW = _TCOLS_FULL // _NW
_SCR_ROWS = _VOCAB // 4
_TAIL = _VOCAB - _TCOLS_FULL * 128
_OUT_ROWS = _N * _HID // 128


_BAT = 4
_STEPS = _TPW // _BAT


def _tr_body(tabT_hbm, tail_hbm, scr_hbm, vsrc_a, vsrc_b, dst_a, dst_b,
             sem_ia, sem_ib, sem_oa, sem_ob):
    wid = lax.axis_index("s") * _NC + lax.axis_index("c")
    tbase = wid * _TPW

    def issue_in(tc, n, vsrc, sem):
        pltpu.async_copy(tabT_hbm.at[:, pl.ds(tc * 128, n * 128)], vsrc, sem)

    def wait_in(tc, n, vsrc, sem):
        pltpu.make_async_copy(
            tabT_hbm.at[:, pl.ds(tc * 128, n * 128)], vsrc, sem).wait()

    def issue_out(tc, n, dst, sem):
        pltpu.async_copy(dst, scr_hbm.at[pl.ds(tc * 32, n * 32)], sem)

    def wait_out(tc, n, dst, sem):
        pltpu.make_async_copy(
            dst, scr_hbm.at[pl.ds(tc * 32, n * 32)], sem).wait()

    iot = lax.iota(jnp.int32, _LANES)

    def transpose(vsrc, dst, n):
        def hstep(h0, _):
            hvec = (h0 + iot) & 31
            wvec = (iot & 3) * _HID + hvec
            qv0 = lax.shift_right_logical(iot, 2)
            for kb in range(n):
                loads = []
                for k in range(kb * 8, kb * 8 + 8):
                    c0 = k * _LANES
                    loads.append(plsc.load_gather(vsrc, [hvec, iot + c0]))
                for j, k in enumerate(range(kb * 8, kb * 8 + 8)):
                    c0 = k * _LANES
                    plsc.store_scatter(dst, [qv0 + (c0 // 4), wvec], loads[j])
            return 0

        lax.fori_loop(0, 32, hstep, 0)

    issue_in(tbase, _BAT, vsrc_a, sem_ia)

    def pair(tp, _):
        tc0 = tbase + 2 * tp * _BAT
        tc1 = tc0 + _BAT
        issue_in(tc1, _BAT, vsrc_b, sem_ib)

        @pl.when(tp > 0)
        def _():
            wait_out(tc0 - 2 * _BAT, _BAT, dst_a, sem_oa)

        wait_in(tc0, _BAT, vsrc_a, sem_ia)
        transpose(vsrc_a, dst_a, _BAT)
        issue_out(tc0, _BAT, dst_a, sem_oa)

        @pl.when(tp < _STEPS // 2 - 1)
        def _():
            issue_in(tc0 + 2 * _BAT, _BAT, vsrc_a, sem_ia)

        @pl.when(tp > 0)
        def _():
            wait_out(tc1 - 2 * _BAT, _BAT, dst_b, sem_ob)

        wait_in(tc1, _BAT, vsrc_b, sem_ib)
        transpose(vsrc_b, dst_b, _BAT)
        issue_out(tc1, _BAT, dst_b, sem_ob)
        return 0

    lax.fori_loop(0, _STEPS // 2, pair, 0)
    wait_out(tbase + (_STEPS - 2) * _BAT, _BAT, dst_a, sem_oa)
    wait_out(tbase + (_STEPS - 1) * _BAT, _BAT, dst_b, sem_ob)

    tlast = tbase + (_STEPS - 1) * _BAT
    pltpu.sync_copy(tabT_hbm.at[:, pl.ds(tlast * 128, _BAT * 128)], vsrc_a)
    transpose(vsrc_a, dst_a, _BAT)
    pltpu.sync_copy(dst_a, scr_hbm.at[pl.ds(tlast * 32, _BAT * 32)])

    @pl.when(wid < 4)
    def _():
        tc = _TCOLS_FULL - 4 + wid
        pltpu.sync_copy(tabT_hbm.at[:, pl.ds(tc * 128, 128)],
                        vsrc_b.at[:, pl.ds(0, 128)])
        transpose(vsrc_b, dst_b, 1)
        pltpu.sync_copy(dst_b.at[pl.ds(0, 32)], scr_hbm.at[pl.ds(tc * 32, 32)])

    @pl.when(wid == 4)
    def _():
        pltpu.sync_copy(tail_hbm, dst_b.at[pl.ds(0, _TAIL // 4)])
        pltpu.sync_copy(dst_b.at[pl.ds(0, _TAIL // 4)],
                        scr_hbm.at[pl.ds(_TCOLS_FULL * 32, _TAIL // 4)])


def _sc_body(inp_hbm, tab_hbm, ft_hbm, out_hbm,
             x_v, ii_v, fi_v, ft_v,
             ri0, ri1, ri2, ri3, ot0, ot1, ot2, ot3,
             sgi0, sgi1, sgi2, sgi3,
             so0, so1, so2, so3):
    wid = lax.axis_index("s") * _NC + lax.axis_index("c")
    base = wid * _BPW * _L
    iot = lax.iota(jnp.int32, _LANES)

    rows_i = (ri0, ri1, ri2, ri3)
    outs = (ot0, ot1, ot2, ot3)
    sem_gi = (sgi0, sgi1, sgi2, sgi3)
    sem_o = (so0, so1, so2, so3)

    pltpu.sync_copy(inp_hbm.at[pl.ds(base, _BPW * _L)], x_v)
    pltpu.sync_copy(ft_hbm, ft_v)

    pos0 = iot * _L

    def idx_chunk(l, _):
        for g in range(_CH // _LANES):
            x = plsc.load_gather(x_v, [pos0 + (g * _LANES * _L + l)])
            ii = x.astype(jnp.int32)
            fr = x - ii.astype(jnp.float32)
            fi = (fr * 100.0).astype(jnp.int32)
            ii_v[l, pl.ds(g * _LANES, _LANES)] = ii
            fi_v[l, pl.ds(g * _LANES, _LANES)] = fi
        return 0

    lax.fori_loop(0, _L, idx_chunk, 0)

    def issue_gathers(l, i):
        pltpu.async_copy(tab_hbm.at[ii_v.at[l]], rows_i[i], sem_gi[i])

    def wait_gathers(l, i):
        pltpu.make_async_copy(tab_hbm.at[ii_v.at[l]], rows_i[i], sem_gi[i]).wait()

    def combine_chunk(l, i):
        ri, ot = rows_i[i], outs[i]

        for k in range(8):
            bvec = iot + k * _LANES
            fvec = fi_v[l, pl.ds(k * _LANES, _LANES)]

            def hstep(h0, _):
                hvec = (h0 + iot) & 31
                a = plsc.load_gather(ri, [bvec, hvec])
                b = plsc.load_gather(ft_v, [fvec, hvec])
                plsc.store_scatter(ot, [hvec, bvec], a + b)
                return 0

            lax.fori_loop(0, 32, hstep, 0)

    def issue_stores(l, i):
        for tr in range(4):
            row0 = l * 1024 + tr * 256 + wid * 8
            pltpu.async_copy(outs[i].at[pl.ds(tr * 8, 8)],
                             out_hbm.at[pl.ds(row0, 8)], sem_o[i])

    def wait_stores(l, i):
        for tr in range(4):
            row0 = l * 1024 + tr * 256 + wid * 8
            pltpu.make_async_copy(outs[i].at[pl.ds(tr * 8, 8)],
                                  out_hbm.at[pl.ds(row0, 8)], sem_o[i]).wait()

    def process(l, i, first):
        if not first:
            @pl.when(l >= _DEPTH)
            def _():
                wait_stores(l - _DEPTH, i)
        wait_gathers(l, i)
        combine_chunk(l, i)
        issue_stores(l, i)

        @pl.when(l + _DEPTH < _L)
        def _():
            issue_gathers(l + _DEPTH, i)

    for i in range(_DEPTH):
        issue_gathers(i, i)

    def quad(qp, _):
        for i in range(_DEPTH):
            process(qp * _DEPTH + i, i, False)
        return 0

    lax.fori_loop(0, _L // _DEPTH, quad, 0)
    for i in range(_L % _DEPTH):
        process(_L - (_L % _DEPTH) + i, i, False)
    for l in range(_L - _DEPTH, _L):
        wait_stores(l, l % _DEPTH)


@functools.partial(jax.jit)
def kernel(input, int_table, float_table):
    mesh = plsc.VectorSubcoreMesh(core_axis_name="c", subcore_axis_name="s")
    flat = input.reshape(_N)

    tr_call = pl.kernel(
        _tr_body,
        out_type=jax.ShapeDtypeStruct((_SCR_ROWS, 128), jnp.float32),
        mesh=mesh,
        compiler_params=pltpu.CompilerParams(
            use_tc_tiling_on_sc=True, needs_layout_passes=False),
        scratch_types=[
            pltpu.VMEM((_HID, _BAT * 128), jnp.float32),
            pltpu.VMEM((_HID, _BAT * 128), jnp.float32),
            pltpu.VMEM((_BAT * 32, 128), jnp.float32),
            pltpu.VMEM((_BAT * 32, 128), jnp.float32),
            pltpu.SemaphoreType.DMA,
            pltpu.SemaphoreType.DMA,
            pltpu.SemaphoreType.DMA,
            pltpu.SemaphoreType.DMA,
        ],
    )
    tail = lax.slice(int_table, (_TCOLS_FULL * 128, 0), (_VOCAB, _HID))
    scr = tr_call(int_table.T, tail.reshape(_TAIL // 4, 128))
    tabp = scr.reshape(_VOCAB, _HID)

    sc_call = pl.kernel(
        _sc_body,
        out_type=jax.ShapeDtypeStruct((_OUT_ROWS, 128), jnp.float32),
        mesh=mesh,
        compiler_params=pltpu.CompilerParams(
            use_tc_tiling_on_sc=False, needs_layout_passes=False),
        scratch_types=(
            [pltpu.VMEM((_BPW * _L,), jnp.float32),
             pltpu.VMEM((_L, _CH), jnp.int32),
             pltpu.VMEM((_L, _CH), jnp.int32),
             pltpu.VMEM((10 ** 2, _HID), jnp.float32)]
            + [pltpu.VMEM((_CH, _HID), jnp.float32)] * 4
            + [pltpu.VMEM((_HID, _CH), jnp.float32)] * 4
            + [pltpu.SemaphoreType.DMA] * 8
        ),
    )
    out_lines = sc_call(flat, tabp, float_table)
    out = (out_lines.reshape(_L, 4, _NW, 8, 128)
           .transpose(2, 4, 0, 1, 3)
           .reshape(_B, _L, _HID))
    return out
